# baseline (speedup 1.0000x reference)
import jax
import jax.numpy as jnp
from jax.experimental import pallas as pl
from jax.experimental.pallas import tpu as pltpu

B, T, D = 4, 4096, 2048
N, L, DIN = 16, 256, 1024


def _body(bref, tref, emb_ref, feats_ref, w_ref, b_ref, out_ref):
    n = pl.program_id(0)
    acc = jnp.dot(feats_ref[0], w_ref[...], preferred_element_type=jnp.float32)
    out_ref[0, 0] = acc + b_ref[...]


def _out_index(n, bref, tref):
    return bref[n], pl.multiple_of(tref[n], L) // L, 0


def kernel(emb, feats, batch_idxs, time_idxs, W, b):
    b2 = b.reshape(1, D)
    embv = emb.reshape(B, T // L, L, D)
    out = pl.pallas_call(
        _body,
        grid_spec=pltpu.PrefetchScalarGridSpec(
            num_scalar_prefetch=2,
            grid=(N,),
            in_specs=[
                pl.BlockSpec(memory_space=pl.ANY),
                pl.BlockSpec((1, L, DIN), lambda n, bref, tref: (n, 0, 0)),
                pl.BlockSpec((DIN, D), lambda n, bref, tref: (0, 0)),
                pl.BlockSpec((1, D), lambda n, bref, tref: (0, 0)),
            ],
            out_specs=pl.BlockSpec((1, 1, L, D),
                                   lambda n, bref, tref: (*_out_index(n, bref, tref), 0)),
        ),
        out_shape=jax.ShapeDtypeStruct((B, T // L, L, D), jnp.float32),
        input_output_aliases={2: 0},
    )(batch_idxs, time_idxs, embv, feats, W, b2)
    return out.reshape(B, T, D)


# speedup vs baseline: 1.3348x; 1.3348x over previous
import jax
import jax.numpy as jnp
from jax.experimental import pallas as pl
from jax.experimental.pallas import tpu as pltpu

B, T, D = 4, 4096, 2048
N, L, DIN = 16, 256, 1024
BT = 1024
S = BT // L


def _slot_match(bi, ti, s, bref, tref):
    covered = None
    seg = jnp.int32(0)
    for n in range(N):
        hit = (bref[n] == bi) & (tref[n] == ti * BT + s * L)
        seg = jnp.where(hit, jnp.int32(n), seg)
        covered = hit if covered is None else (covered | hit)
    return covered, seg


def _feats_index(s):
    def index(bi, ti, bref, tref):
        _, seg = _slot_match(bi, ti, s, bref, tref)
        return seg, 0, 0
    return index


def _emb_index(s):
    def index(bi, ti, bref, tref):
        covered, _ = _slot_match(bi, ti, s, bref, tref)
        lin = jnp.where(covered, s, (bi * (T // BT) + ti) * S + s)
        return lin // (S * (T // BT)), lin % (S * (T // BT)), 0, 0
    return index


def _body(bref, tref, e0, e1, e2, e3, f0, f1, f2, f3, w_ref, b_ref, out_ref):
    bi = pl.program_id(0)
    ti = pl.program_id(1)
    erefs = (e0, e1, e2, e3)
    frefs = (f0, f1, f2, f3)
    for s in range(S):
        covered, _ = _slot_match(bi, ti, s, bref, tref)

        @pl.when(covered)
        def _(s=s):
            acc = jnp.dot(frefs[s][0], w_ref[...],
                          preferred_element_type=jnp.float32)
            out_ref[0, s * L:(s + 1) * L, :] = acc + b_ref[...]

        @pl.when(jnp.logical_not(covered))
        def _(s=s):
            out_ref[0, s * L:(s + 1) * L, :] = erefs[s][0, 0]


def kernel(emb, feats, batch_idxs, time_idxs, W, b):
    b2 = b.reshape(1, D)
    embv = emb.reshape(B, (T // BT) * S, L, D)
    grid = (B, T // BT)
    emb_specs = [pl.BlockSpec((1, 1, L, D), _emb_index(s)) for s in range(S)]
    feats_specs = [pl.BlockSpec((1, L, DIN), _feats_index(s))
                   for s in range(S)]
    out = pl.pallas_call(
        _body,
        grid_spec=pltpu.PrefetchScalarGridSpec(
            num_scalar_prefetch=2,
            grid=grid,
            in_specs=[
                *emb_specs,
                *feats_specs,
                pl.BlockSpec((DIN, D), lambda bi, ti, bref, tref: (0, 0)),
                pl.BlockSpec((1, D), lambda bi, ti, bref, tref: (0, 0)),
            ],
            out_specs=pl.BlockSpec((1, BT, D),
                                   lambda bi, ti, bref, tref: (bi, ti, 0)),
        ),
        out_shape=jax.ShapeDtypeStruct((B, T, D), jnp.float32),
    )(batch_idxs, time_idxs, embv, embv, embv, embv,
      feats, feats, feats, feats, W, b2)
    return out
